# initial kernel scaffold (unmeasured)
import jax
import jax.numpy as jnp
from jax import lax
from jax.experimental import pallas as pl
from jax.experimental.pallas import tpu as pltpu

Y = 2
T = 1024
TL = T // Y
D = 1024
E = 8
EL = E // Y
F = 2048
FT = 512
NFT = F // FT


def _peer():
    return (lax.axis_index("x"), 1 - lax.axis_index("y"), lax.axis_index("z"))


def _peer_barrier():
    barrier = pltpu.get_barrier_semaphore()
    pl.semaphore_signal(
        barrier, inc=1, device_id=_peer(), device_id_type=pl.DeviceIdType.MESH
    )
    pl.semaphore_wait(barrier, 1)


def _exchange_body(x_ref, r_ref, xf_ref, rf_ref, send_sems, recv_sems):
    my_y = lax.axis_index("y")
    peer = _peer()
    _peer_barrier()

    xf_ref[pl.ds(my_y * TL, TL), :] = x_ref[...]
    rf_ref[my_y, :, :] = r_ref[...]

    rdma_x = pltpu.make_async_remote_copy(
        src_ref=x_ref,
        dst_ref=xf_ref.at[pl.ds(my_y * TL, TL), :],
        send_sem=send_sems.at[0],
        recv_sem=recv_sems.at[0],
        device_id=peer,
        device_id_type=pl.DeviceIdType.MESH,
    )
    rdma_r = pltpu.make_async_remote_copy(
        src_ref=r_ref,
        dst_ref=rf_ref.at[my_y],
        send_sem=send_sems.at[1],
        recv_sem=recv_sems.at[1],
        device_id=peer,
        device_id_type=pl.DeviceIdType.MESH,
    )
    rdma_x.start()
    rdma_r.start()
    rdma_x.wait()
    rdma_r.wait()


def _ffn_body(x_ref, w1_ref, w2_ref, w_ref, out_ref):
    e = pl.program_id(0)
    f = pl.program_id(1)

    @pl.when(jnp.logical_and(e == 0, f == 0))
    def _():
        out_ref[...] = jnp.zeros_like(out_ref)

    h = lax.dot_general(
        x_ref[...], w1_ref[0],
        (((1,), (0,)), ((), ())),
        preferred_element_type=jnp.float32,
    )
    h = jnp.maximum(h, 0.0) * w_ref[...]
    out_ref[...] += lax.dot_general(
        h, w2_ref[0],
        (((1,), (0,)), ((), ())),
        preferred_element_type=jnp.float32,
    )


def _combine_body(p_ref, o_ref, recv_buf, send_sem, recv_sem):
    my_y = lax.axis_index("y")
    peer = _peer()
    _peer_barrier()

    rdma = pltpu.make_async_remote_copy(
        src_ref=p_ref.at[pl.ds((1 - my_y) * TL, TL), :],
        dst_ref=recv_buf,
        send_sem=send_sem,
        recv_sem=recv_sem,
        device_id=peer,
        device_id_type=pl.DeviceIdType.MESH,
    )
    rdma.start()
    rdma.wait()

    o_ref[...] = p_ref[pl.ds(my_y * TL, TL), :] + recv_buf[...]


def kernel(x, router, W1, W2):
    x_full, rf = pl.pallas_call(
        _exchange_body,
        out_shape=[
            jax.ShapeDtypeStruct((T, D), jnp.float32),
            jax.ShapeDtypeStruct((Y, D, EL), jnp.float32),
        ],
        in_specs=[
            pl.BlockSpec(memory_space=pltpu.VMEM),
            pl.BlockSpec(memory_space=pltpu.VMEM),
        ],
        out_specs=[
            pl.BlockSpec(memory_space=pltpu.VMEM),
            pl.BlockSpec(memory_space=pltpu.VMEM),
        ],
        scratch_shapes=[
            pltpu.SemaphoreType.DMA((2,)),
            pltpu.SemaphoreType.DMA((2,)),
        ],
        compiler_params=pltpu.CompilerParams(collective_id=0),
    )(x, router)

    router_full = jnp.concatenate([rf[0], rf[1]], axis=1)
    gates = x_full @ router_full
    vals, idx = lax.top_k(gates, 2)
    w_hi = jax.nn.sigmoid(vals[:, 0] - vals[:, 1])
    w_lo = 1.0 - w_hi
    eids = jnp.arange(E)[None, :]
    w_all = (
        w_hi[:, None] * (idx[:, 0:1] == eids)
        + w_lo[:, None] * (idx[:, 1:2] == eids)
    ).astype(jnp.float32)
    my_y = lax.axis_index("y")
    w_loc = lax.dynamic_slice(w_all, (0, my_y * EL), (T, EL))

    partial = pl.pallas_call(
        _ffn_body,
        grid=(EL, NFT),
        in_specs=[
            pl.BlockSpec((T, D), lambda e, f: (0, 0)),
            pl.BlockSpec((1, D, FT), lambda e, f: (e, 0, f)),
            pl.BlockSpec((1, FT, D), lambda e, f: (e, f, 0)),
            pl.BlockSpec((T, 1), lambda e, f: (0, e)),
        ],
        out_specs=pl.BlockSpec((T, D), lambda e, f: (0, 0)),
        out_shape=jax.ShapeDtypeStruct((T, D), jnp.float32),
        compiler_params=pltpu.CompilerParams(
            dimension_semantics=("arbitrary", "arbitrary"),
        ),
    )(x_full, W1, W2, w_loc)

    out = pl.pallas_call(
        _combine_body,
        out_shape=jax.ShapeDtypeStruct((TL, D), jnp.float32),
        in_specs=[pl.BlockSpec(memory_space=pltpu.VMEM)],
        out_specs=pl.BlockSpec(memory_space=pltpu.VMEM),
        scratch_shapes=[
            pltpu.VMEM((TL, D), jnp.float32),
            pltpu.SemaphoreType.DMA,
            pltpu.SemaphoreType.DMA,
        ],
        compiler_params=pltpu.CompilerParams(collective_id=1),
    )(partial)

    return out


# baseline (device time: 111481 ns/iter reference)
import jax
import jax.numpy as jnp
from jax import lax
from jax.experimental import pallas as pl
from jax.experimental.pallas import tpu as pltpu

Y = 2
T = 1024
TL = T // Y
D = 1024
E = 8
EL = E // Y
F = 2048
FT = 512
NFT = F // FT


def _peer():
    return (lax.axis_index("x"), 1 - lax.axis_index("y"), lax.axis_index("z"))


def _peer_barrier():
    barrier = pltpu.get_barrier_semaphore()
    pl.semaphore_signal(
        barrier, inc=1, device_id=_peer(), device_id_type=pl.DeviceIdType.MESH
    )
    pl.semaphore_wait(barrier, 1)


def _exchange_body(x_ref, r_ref, xf_ref, rf_ref, send_sems, recv_sems):
    my_y = lax.axis_index("y")
    peer = _peer()
    _peer_barrier()

    xf_ref[pl.ds(my_y * TL, TL), :] = x_ref[...]
    rf_ref[my_y, :, :] = r_ref[...]

    rdma_x = pltpu.make_async_remote_copy(
        src_ref=x_ref,
        dst_ref=xf_ref.at[pl.ds(my_y * TL, TL), :],
        send_sem=send_sems.at[0],
        recv_sem=recv_sems.at[0],
        device_id=peer,
        device_id_type=pl.DeviceIdType.MESH,
    )
    rdma_r = pltpu.make_async_remote_copy(
        src_ref=r_ref,
        dst_ref=rf_ref.at[my_y],
        send_sem=send_sems.at[1],
        recv_sem=recv_sems.at[1],
        device_id=peer,
        device_id_type=pl.DeviceIdType.MESH,
    )
    rdma_x.start()
    rdma_r.start()
    rdma_x.wait()
    rdma_r.wait()


def _ffn_body(x_ref, w1_ref, w2_ref, w_ref, out_ref):
    e = pl.program_id(0)
    f = pl.program_id(1)

    @pl.when(jnp.logical_and(e == 0, f == 0))
    def _():
        out_ref[...] = jnp.zeros_like(out_ref)

    sel = (lax.broadcasted_iota(jnp.int32, (EL, 1), 0) == e).astype(jnp.float32)
    w_col = lax.dot_general(
        w_ref[...], sel,
        (((1,), (0,)), ((), ())),
        preferred_element_type=jnp.float32,
    )
    h = lax.dot_general(
        x_ref[...], w1_ref[0],
        (((1,), (0,)), ((), ())),
        preferred_element_type=jnp.float32,
    )
    h = jnp.maximum(h, 0.0) * w_col
    out_ref[...] += lax.dot_general(
        h, w2_ref[0],
        (((1,), (0,)), ((), ())),
        preferred_element_type=jnp.float32,
    )


def _combine_body(p_ref, o_ref, recv_buf, send_sem, recv_sem):
    my_y = lax.axis_index("y")
    peer = _peer()
    _peer_barrier()

    rdma = pltpu.make_async_remote_copy(
        src_ref=p_ref.at[pl.ds((1 - my_y) * TL, TL), :],
        dst_ref=recv_buf,
        send_sem=send_sem,
        recv_sem=recv_sem,
        device_id=peer,
        device_id_type=pl.DeviceIdType.MESH,
    )
    rdma.start()
    rdma.wait()

    o_ref[...] = p_ref[pl.ds(my_y * TL, TL), :] + recv_buf[...]


def kernel(x, router, W1, W2):
    x_full, rf = pl.pallas_call(
        _exchange_body,
        out_shape=[
            jax.ShapeDtypeStruct((T, D), jnp.float32),
            jax.ShapeDtypeStruct((Y, D, EL), jnp.float32),
        ],
        in_specs=[
            pl.BlockSpec(memory_space=pltpu.VMEM),
            pl.BlockSpec(memory_space=pltpu.VMEM),
        ],
        out_specs=[
            pl.BlockSpec(memory_space=pltpu.VMEM),
            pl.BlockSpec(memory_space=pltpu.VMEM),
        ],
        scratch_shapes=[
            pltpu.SemaphoreType.DMA((2,)),
            pltpu.SemaphoreType.DMA((2,)),
        ],
        compiler_params=pltpu.CompilerParams(collective_id=0),
    )(x, router)

    router_full = jnp.concatenate([rf[0], rf[1]], axis=1)
    gates = jnp.dot(
        x_full, router_full, precision=lax.Precision.HIGHEST
    )
    vals, idx = lax.top_k(gates, 2)
    w_hi = jax.nn.sigmoid(vals[:, 0] - vals[:, 1])
    w_lo = 1.0 - w_hi
    eids = jnp.arange(E)[None, :]
    w_all = (
        w_hi[:, None] * (idx[:, 0:1] == eids)
        + w_lo[:, None] * (idx[:, 1:2] == eids)
    ).astype(jnp.float32)
    my_y = lax.axis_index("y")
    w_loc = lax.dynamic_slice(w_all, (0, my_y * EL), (T, EL))

    partial = pl.pallas_call(
        _ffn_body,
        grid=(EL, NFT),
        in_specs=[
            pl.BlockSpec((T, D), lambda e, f: (0, 0)),
            pl.BlockSpec((1, D, FT), lambda e, f: (e, 0, f)),
            pl.BlockSpec((1, FT, D), lambda e, f: (e, f, 0)),
            pl.BlockSpec((T, EL), lambda e, f: (0, 0)),
        ],
        out_specs=pl.BlockSpec((T, D), lambda e, f: (0, 0)),
        out_shape=jax.ShapeDtypeStruct((T, D), jnp.float32),
        compiler_params=pltpu.CompilerParams(
            dimension_semantics=("arbitrary", "arbitrary"),
        ),
    )(x_full, W1, W2, w_loc)

    out = pl.pallas_call(
        _combine_body,
        out_shape=jax.ShapeDtypeStruct((TL, D), jnp.float32),
        in_specs=[pl.BlockSpec(memory_space=pltpu.VMEM)],
        out_specs=pl.BlockSpec(memory_space=pltpu.VMEM),
        scratch_shapes=[
            pltpu.VMEM((TL, D), jnp.float32),
            pltpu.SemaphoreType.DMA,
            pltpu.SemaphoreType.DMA,
        ],
        compiler_params=pltpu.CompilerParams(collective_id=1),
    )(partial)

    return out


# device time: 111281 ns/iter; 1.0018x vs baseline; 1.0018x over previous
import jax
import jax.numpy as jnp
from jax import lax
from jax.experimental import pallas as pl
from jax.experimental.pallas import tpu as pltpu

Y = 2
T = 1024
TL = T // Y
D = 1024
E = 8
EL = E // Y
F = 2048
FT = 512
NFT = F // FT


def _peer():
    return (lax.axis_index("x"), 1 - lax.axis_index("y"), lax.axis_index("z"))


def _peer_barrier():
    barrier = pltpu.get_barrier_semaphore()
    pl.semaphore_signal(
        barrier, inc=1, device_id=_peer(), device_id_type=pl.DeviceIdType.MESH
    )
    pl.semaphore_wait(barrier, 1)


def _exchange_body(x_ref, r_ref, xf_ref, rf_ref, send_sems, recv_sems):
    my_y = lax.axis_index("y")
    peer = _peer()
    _peer_barrier()

    xf_ref[pl.ds(my_y * TL, TL), :] = x_ref[...]
    rf_ref[my_y, :, :] = r_ref[...]

    rdma_x = pltpu.make_async_remote_copy(
        src_ref=x_ref,
        dst_ref=xf_ref.at[pl.ds(my_y * TL, TL), :],
        send_sem=send_sems.at[0],
        recv_sem=recv_sems.at[0],
        device_id=peer,
        device_id_type=pl.DeviceIdType.MESH,
    )
    rdma_r = pltpu.make_async_remote_copy(
        src_ref=r_ref,
        dst_ref=rf_ref.at[my_y],
        send_sem=send_sems.at[1],
        recv_sem=recv_sems.at[1],
        device_id=peer,
        device_id_type=pl.DeviceIdType.MESH,
    )
    rdma_x.start()
    rdma_r.start()
    rdma_x.wait()
    rdma_r.wait()


def _ffn_body(x_ref, w1_ref, w2_ref, w_ref, out_ref):
    e = pl.program_id(0)
    f = pl.program_id(1)

    @pl.when(jnp.logical_and(e == 0, f == 0))
    def _():
        out_ref[...] = jnp.zeros_like(out_ref)

    sel = (lax.broadcasted_iota(jnp.int32, (EL, 1), 0) == e).astype(jnp.float32)
    w_col = lax.dot_general(
        w_ref[...], sel,
        (((1,), (0,)), ((), ())),
        preferred_element_type=jnp.float32,
    )
    h = lax.dot_general(
        x_ref[...].astype(jnp.bfloat16), w1_ref[0].astype(jnp.bfloat16),
        (((1,), (0,)), ((), ())),
        preferred_element_type=jnp.float32,
    )
    h = jnp.maximum(h, 0.0) * w_col
    out_ref[...] += lax.dot_general(
        h.astype(jnp.bfloat16), w2_ref[0].astype(jnp.bfloat16),
        (((1,), (0,)), ((), ())),
        preferred_element_type=jnp.float32,
    )


def _combine_body(p_ref, o_ref, recv_buf, send_sem, recv_sem):
    my_y = lax.axis_index("y")
    peer = _peer()
    _peer_barrier()

    rdma = pltpu.make_async_remote_copy(
        src_ref=p_ref.at[pl.ds((1 - my_y) * TL, TL), :],
        dst_ref=recv_buf,
        send_sem=send_sem,
        recv_sem=recv_sem,
        device_id=peer,
        device_id_type=pl.DeviceIdType.MESH,
    )
    rdma.start()
    rdma.wait()

    o_ref[...] = p_ref[pl.ds(my_y * TL, TL), :] + recv_buf[...]


def kernel(x, router, W1, W2):
    x_full, rf = pl.pallas_call(
        _exchange_body,
        out_shape=[
            jax.ShapeDtypeStruct((T, D), jnp.float32),
            jax.ShapeDtypeStruct((Y, D, EL), jnp.float32),
        ],
        in_specs=[
            pl.BlockSpec(memory_space=pltpu.VMEM),
            pl.BlockSpec(memory_space=pltpu.VMEM),
        ],
        out_specs=[
            pl.BlockSpec(memory_space=pltpu.VMEM),
            pl.BlockSpec(memory_space=pltpu.VMEM),
        ],
        scratch_shapes=[
            pltpu.SemaphoreType.DMA((2,)),
            pltpu.SemaphoreType.DMA((2,)),
        ],
        compiler_params=pltpu.CompilerParams(collective_id=0),
    )(x, router)

    router_full = jnp.concatenate([rf[0], rf[1]], axis=1)
    gates = jnp.dot(
        x_full, router_full, precision=lax.Precision.HIGHEST
    )
    vals, idx = lax.top_k(gates, 2)
    w_hi = jax.nn.sigmoid(vals[:, 0] - vals[:, 1])
    w_lo = 1.0 - w_hi
    eids = jnp.arange(E)[None, :]
    w_all = (
        w_hi[:, None] * (idx[:, 0:1] == eids)
        + w_lo[:, None] * (idx[:, 1:2] == eids)
    ).astype(jnp.float32)
    my_y = lax.axis_index("y")
    w_loc = lax.dynamic_slice(w_all, (0, my_y * EL), (T, EL))

    partial = pl.pallas_call(
        _ffn_body,
        grid=(EL, NFT),
        in_specs=[
            pl.BlockSpec((T, D), lambda e, f: (0, 0)),
            pl.BlockSpec((1, D, FT), lambda e, f: (e, 0, f)),
            pl.BlockSpec((1, FT, D), lambda e, f: (e, f, 0)),
            pl.BlockSpec((T, EL), lambda e, f: (0, 0)),
        ],
        out_specs=pl.BlockSpec((T, D), lambda e, f: (0, 0)),
        out_shape=jax.ShapeDtypeStruct((T, D), jnp.float32),
        compiler_params=pltpu.CompilerParams(
            dimension_semantics=("arbitrary", "arbitrary"),
        ),
    )(x_full, W1, W2, w_loc)

    out = pl.pallas_call(
        _combine_body,
        out_shape=jax.ShapeDtypeStruct((TL, D), jnp.float32),
        in_specs=[pl.BlockSpec(memory_space=pltpu.VMEM)],
        out_specs=pl.BlockSpec(memory_space=pltpu.VMEM),
        scratch_shapes=[
            pltpu.VMEM((TL, D), jnp.float32),
            pltpu.SemaphoreType.DMA,
            pltpu.SemaphoreType.DMA,
        ],
        compiler_params=pltpu.CompilerParams(collective_id=1),
    )(partial)

    return out


# device time: 87472 ns/iter; 1.2745x vs baseline; 1.2722x over previous
import jax
import jax.numpy as jnp
from jax import lax
from jax.experimental import pallas as pl
from jax.experimental.pallas import tpu as pltpu

Y = 2
T = 1024
TL = T // Y
D = 1024
E = 8
EL = E // Y
F = 2048
FT = 1024
NFT = F // FT


def _peer():
    return (lax.axis_index("x"), 1 - lax.axis_index("y"), lax.axis_index("z"))


def _body(x_ref, r_ref, w1_ref, w2_ref, o_ref,
          xf_ref, rf_ref, w_ref, acc_ref, xb_ref, csend_ref, crecv_ref,
          send_sems, recv_sems):
    e = pl.program_id(0)
    f = pl.program_id(1)
    my_y = lax.axis_index("y")
    peer = _peer()

    @pl.when(jnp.logical_and(e == 0, f == 0))
    def _exchange():
        barrier = pltpu.get_barrier_semaphore()
        pl.semaphore_signal(
            barrier, inc=1, device_id=peer,
            device_id_type=pl.DeviceIdType.MESH,
        )
        pl.semaphore_wait(barrier, 1)

        xb_ref[...] = x_ref[...].astype(jnp.bfloat16)
        xf_ref[pl.ds(my_y * TL, TL), :] = xb_ref[...]
        rf_ref[my_y, :, :] = r_ref[...]

        rdma_x = pltpu.make_async_remote_copy(
            src_ref=xb_ref,
            dst_ref=xf_ref.at[pl.ds(my_y * TL, TL), :],
            send_sem=send_sems.at[0],
            recv_sem=recv_sems.at[0],
            device_id=peer,
            device_id_type=pl.DeviceIdType.MESH,
        )
        rdma_r = pltpu.make_async_remote_copy(
            src_ref=r_ref,
            dst_ref=rf_ref.at[my_y],
            send_sem=send_sems.at[1],
            recv_sem=recv_sems.at[1],
            device_id=peer,
            device_id_type=pl.DeviceIdType.MESH,
        )
        rdma_x.start()
        rdma_r.start()
        rdma_r.wait()

        router_full = jnp.concatenate([rf_ref[0], rf_ref[1]], axis=1)
        gates = lax.dot_general(
            x_ref[...], router_full,
            (((1,), (0,)), ((), ())),
            precision=lax.Precision.HIGHEST,
            preferred_element_type=jnp.float32,
        )
        eids = lax.broadcasted_iota(jnp.int32, (TL, E), 1)
        i1 = jnp.argmax(gates, axis=1, keepdims=True)
        v1 = jnp.max(gates, axis=1, keepdims=True)
        masked = jnp.where(eids == i1, -jnp.inf, gates)
        i2 = jnp.argmax(masked, axis=1, keepdims=True)
        v2 = jnp.max(masked, axis=1, keepdims=True)
        w_hi = 1.0 / (1.0 + jnp.exp(v2 - v1))
        w_lo = 1.0 - w_hi
        w_rows = (
            jnp.where(eids == i1, w_hi, 0.0) + jnp.where(eids == i2, w_lo, 0.0)
        )
        w_ref[pl.ds(my_y * TL, TL), :] = w_rows

        rdma_w = pltpu.make_async_remote_copy(
            src_ref=w_ref.at[pl.ds(my_y * TL, TL), :],
            dst_ref=w_ref.at[pl.ds(my_y * TL, TL), :],
            send_sem=send_sems.at[2],
            recv_sem=recv_sems.at[2],
            device_id=peer,
            device_id_type=pl.DeviceIdType.MESH,
        )
        rdma_w.start()
        rdma_w.wait()
        rdma_x.wait()

        acc_ref[...] = jnp.zeros_like(acc_ref)

    sel = (
        lax.broadcasted_iota(jnp.int32, (E, 1), 0) == (my_y * EL + e)
    ).astype(jnp.float32)
    w_col = lax.dot_general(
        w_ref[...], sel,
        (((1,), (0,)), ((), ())),
        preferred_element_type=jnp.float32,
    )
    h = lax.dot_general(
        xf_ref[...], w1_ref[0].astype(jnp.bfloat16),
        (((1,), (0,)), ((), ())),
        preferred_element_type=jnp.float32,
    )
    h = jnp.maximum(h, 0.0) * w_col
    acc_ref[...] += lax.dot_general(
        h.astype(jnp.bfloat16), w2_ref[0].astype(jnp.bfloat16),
        (((1,), (0,)), ((), ())),
        preferred_element_type=jnp.float32,
    )

    @pl.when(jnp.logical_and(e == EL - 1, f == NFT - 1))
    def _combine():
        csend_ref[...] = acc_ref[pl.ds((1 - my_y) * TL, TL), :].astype(
            jnp.bfloat16
        )
        rdma_c = pltpu.make_async_remote_copy(
            src_ref=csend_ref,
            dst_ref=crecv_ref,
            send_sem=send_sems.at[1],
            recv_sem=recv_sems.at[1],
            device_id=peer,
            device_id_type=pl.DeviceIdType.MESH,
        )
        rdma_c.start()
        rdma_c.wait()
        o_ref[...] = acc_ref[pl.ds(my_y * TL, TL), :] + crecv_ref[...].astype(
            jnp.float32
        )


def kernel(x, router, W1, W2):
    return pl.pallas_call(
        _body,
        grid=(EL, NFT),
        in_specs=[
            pl.BlockSpec((TL, D), lambda e, f: (0, 0)),
            pl.BlockSpec((D, EL), lambda e, f: (0, 0)),
            pl.BlockSpec((1, D, FT), lambda e, f: (e, 0, f)),
            pl.BlockSpec((1, FT, D), lambda e, f: (e, f, 0)),
        ],
        out_specs=pl.BlockSpec((TL, D), lambda e, f: (0, 0)),
        out_shape=jax.ShapeDtypeStruct((TL, D), jnp.float32),
        scratch_shapes=[
            pltpu.VMEM((T, D), jnp.bfloat16),
            pltpu.VMEM((Y, D, EL), jnp.float32),
            pltpu.VMEM((T, E), jnp.float32),
            pltpu.VMEM((T, D), jnp.float32),
            pltpu.VMEM((TL, D), jnp.bfloat16),
            pltpu.VMEM((TL, D), jnp.bfloat16),
            pltpu.VMEM((TL, D), jnp.bfloat16),
            pltpu.SemaphoreType.DMA((3,)),
            pltpu.SemaphoreType.DMA((3,)),
        ],
        compiler_params=pltpu.CompilerParams(
            collective_id=0,
            dimension_semantics=("arbitrary", "arbitrary"),
        ),
    )(x, router, W1, W2)
